# baseline (speedup 1.0000x reference)
import functools

import jax
import jax.numpy as jnp
from jax import lax
from jax.experimental import pallas as pl
from jax.experimental.pallas import tpu as pltpu
from jax.experimental.pallas import tpu_sc as plsc

H = 64
NIN = 26
NCTX = 20
RPI = 48
C = 4
ROWS = RPI * C
NBUF = 4
PSW = 32
L = 16
NQ = H // L


@functools.lru_cache(maxsize=None)
def _make_sc_scores(B: int, V: int):
    info = plsc.get_sparse_core_info()
    NC, NS = info.num_cores, info.num_subcores
    NW = NC * NS
    assert B % (NW * C) == 0
    BPW = B // NW
    NCH = BPW // C

    mesh = plsc.VectorSubcoreMesh(core_axis_name="c", subcore_axis_name="s")

    @functools.partial(
        pl.kernel,
        mesh=mesh,
        compiler_params=pltpu.CompilerParams(
            needs_layout_passes=False, use_tc_tiling_on_sc=False),
        out_type=jax.ShapeDtypeStruct((B, PSW), jnp.float32),
        scratch_types=[
            pltpu.VMEM((NCH, ROWS), jnp.int32),
            pltpu.VMEM((NBUF, ROWS // 4, 4 * H), jnp.float32),
            pltpu.VMEM((BPW, PSW), jnp.float32),
            pltpu.SemaphoreType.DMA,
            pltpu.SemaphoreType.DMA,
            pltpu.SemaphoreType.DMA,
            pltpu.SemaphoreType.DMA,
        ],
    )
    def sc_scores(idx_hbm, table_hbm, ps_hbm, idx_v, rows_v, ps_v, s0, s1, s2, s3):
        sems = [s0, s1, s2, s3]
        wid = lax.axis_index("s") * NC + lax.axis_index("c")
        pltpu.sync_copy(idx_hbm.at[pl.ds(wid * NCH, NCH)], idx_v)

        def gather(j, ch):
            return pltpu.make_async_copy(
                table_hbm.at[idx_v.at[ch, pl.ds(0, ROWS // 4)]],
                rows_v.at[j], sems[j])

        for j in range(NBUF):
            gather(j, j).start()

        lane_iota = lax.iota(jnp.int32, 16)

        def process(jd, ch):
            for k in range(C):
                base = k * RPI
                iv = [idx_v[ch, pl.ds(base + 16 * t, 16)] for t in range(RPI // 16)]
                mv = [jnp.where(v > 0, 1.0, 0.0) for v in iv]

                ctx = [jnp.zeros((L,), jnp.float32)] * NQ
                for w in range(NCTX):
                    r = base + NIN + w
                    m = mv[(NIN + w) // 16][(NIN + w) % 16]
                    for q in range(NQ):
                        ctx[q] = ctx[q] + rows_v[jd, r, pl.ds(q * L, L)] * m
                ctx = [cq * (1.0 / NCTX) for cq in ctx]

                ps0 = jnp.zeros((L,), jnp.float32)
                ps1 = jnp.zeros((L,), jnp.float32)
                for nn in range(NIN):
                    r = base + nn
                    t = rows_v[jd, r, pl.ds(0, L)] * ctx[0]
                    for q in range(1, NQ):
                        t = t + rows_v[jd, r, pl.ds(q * L, L)] * ctx[q]
                    p = jnp.sum(t) * mv[nn // 16][nn % 16]
                    if nn < 16:
                        ps0 = jnp.where(lane_iota == nn, p, ps0)
                    else:
                        ps1 = jnp.where(lane_iota == (nn - 16), p, ps1)
                il = ch * C + k
                ps_v[il, pl.ds(0, L)] = ps0
                ps_v[il, pl.ds(L, L)] = ps1

        def outer(ch, _):
            jd = lax.rem(ch, NBUF)
            for j in range(NBUF):
                @pl.when(jd == j)
                def _():
                    gather(j, ch).wait()
            if True:
                pass
            else:
                process(jd, ch)

            @pl.when(ch + NBUF < NCH)
            def _():
                for j in range(NBUF):
                    @pl.when(jd == j)
                    def _():
                        gather(j, ch + NBUF).start()
            return 0

        lax.fori_loop(0, NCH, outer, 0)
        pltpu.sync_copy(ps_v, ps_hbm.at[pl.ds(wid * BPW, BPW)])

    return sc_scores


def _loss_body(ps_ref, out_ref):
    x = ps_ref[...]
    col = lax.broadcasted_iota(jnp.int32, x.shape, 1)
    xm = jnp.where(col < NIN, x, -1e30)
    m = jnp.max(xm, axis=1, keepdims=True)
    se = jnp.sum(jnp.exp(xm - m), axis=1, keepdims=True)
    lse = m + jnp.log(se)
    out_ref[...] = jnp.mean(lse - x[:, 0:1]).reshape(1, 1)


def kernel(targets, contexts, negtives, wordemb):
    B = targets.shape[0]
    V = wordemb.shape[0]
    idx_all = jnp.concatenate(
        [
            targets.astype(jnp.int32).reshape(B, 1),
            negtives.astype(jnp.int32).reshape(B, -1),
            contexts.astype(jnp.int32).reshape(B, -1),
            jnp.zeros((B, RPI - NIN - NCTX), jnp.int32),
        ],
        axis=1,
    ).reshape(B // C, ROWS)
    idx_all = jnp.minimum(idx_all, V // 4 - 1)
    ps = _make_sc_scores(B, V)(idx_all, wordemb.reshape(V // 4, 4 * H))
    loss = pl.pallas_call(
        _loss_body,
        out_shape=jax.ShapeDtypeStruct((1, 1), jnp.float32),
    )(ps)
    return loss[0, 0]


# speedup vs baseline: 2.9140x; 2.9140x over previous
import functools

import jax
import jax.numpy as jnp
from jax import lax
from jax.experimental import pallas as pl
from jax.experimental.pallas import tpu as pltpu
from jax.experimental.pallas import tpu_sc as plsc

H = 64
HP = 128
NIN = 26
NCTX = 20
RPI = 48
C = 8
ROWS = RPI * C
NBUF = 2
PSW = 32
L = 16
NQ = H // L


@functools.lru_cache(maxsize=None)
def _make_sc_scores(B: int, V: int):
    info = plsc.get_sparse_core_info()
    NC, NS = info.num_cores, info.num_subcores
    NW = NC * NS
    assert B % (NW * C) == 0
    BPW = B // NW
    NCH = BPW // C
    IRW = BPW * RPI // 128
    ORW = BPW * PSW // 128

    mesh = plsc.VectorSubcoreMesh(core_axis_name="c", subcore_axis_name="s")

    @functools.partial(
        pl.kernel,
        mesh=mesh,
        compiler_params=pltpu.CompilerParams(needs_layout_passes=False),
        out_type=jax.ShapeDtypeStruct((B * PSW // 128, 128), jnp.float32),
        scratch_types=[
            pltpu.VMEM((BPW * RPI // 128, 128), jnp.int32),
            pltpu.VMEM((NBUF, ROWS, HP), jnp.float32),
            pltpu.VMEM((NBUF, C * PSW // 128, 128), jnp.float32),
            pltpu.SemaphoreType.DMA,
            pltpu.SemaphoreType.DMA,
            pltpu.SemaphoreType.DMA,
            pltpu.SemaphoreType.DMA,
        ],
    )
    def sc_scores(idx_hbm, table_hbm, ps_hbm,
                  idx_v, rows_v, psb_v, s0, s1, p0, p1):
        sems = [s0, s1]
        psems = [p0, p1]
        OPC = C * PSW // 128
        wid = lax.axis_index("s") * NC + lax.axis_index("c")
        pltpu.sync_copy(idx_hbm.at[pl.ds(wid * IRW, IRW)], idx_v)

        def iv_load(ch, t):
            f = 16 * t
            return idx_v[ch * (ROWS // 128) + f // 128, pl.ds(f % 128, L)]

        def gather_start(j, ch):
            for t in range(ROWS // L):
                ivp = lax.shift_right_logical(iv_load(ch, t), 1)
                pltpu.make_async_copy(
                    table_hbm.at[ivp],
                    rows_v.at[j, pl.ds(t * L, L)], sems[j]).start()

        def gather_wait(j):
            pltpu.make_async_copy(
                table_hbm.at[pl.ds(0, ROWS)], rows_v.at[j], sems[j]).wait()

        def ps_wait(jr):
            pltpu.make_async_copy(
                ps_hbm.at[pl.ds(0, OPC)], psb_v.at[jr], psems[jr]).wait()

        def ps_start(jr, ch):
            pltpu.make_async_copy(
                psb_v.at[jr],
                ps_hbm.at[pl.ds(wid * ORW + ch * OPC, OPC)],
                psems[jr]).start()

        for j in range(NBUF):
            gather_start(j, j)

        lane_iota = lax.iota(jnp.int32, 16)

        def process(jd, jr, ch):
            for kk in range(C):
                base = kk * RPI
                iv = [iv_load(ch, (base + 16 * t) // 16)
                      for t in range(RPI // L)]
                mv = [jnp.where(v > 0, 1.0, 0.0) for v in iv]
                pv = [v & 1 for v in iv]

                def half(r, rr):
                    off = pv[rr // 16][rr % 16] * H
                    return [rows_v[jd, r, pl.ds(off + q * L, L)]
                            for q in range(NQ)]

                ctx = [jnp.zeros((L,), jnp.float32)] * NQ
                for w in range(NCTX):
                    rr = NIN + w
                    m = mv[rr // 16][rr % 16]
                    hv = half(base + rr, rr)
                    for q in range(NQ):
                        ctx[q] = ctx[q] + hv[q] * m
                ctx = [cq * (1.0 / NCTX) for cq in ctx]

                ps0 = jnp.zeros((L,), jnp.float32)
                ps1 = jnp.zeros((L,), jnp.float32)
                for nn in range(NIN):
                    hv = half(base + nn, nn)
                    t = hv[0] * ctx[0]
                    for q in range(1, NQ):
                        t = t + hv[q] * ctx[q]
                    p = jnp.sum(t) * mv[nn // 16][nn % 16]
                    if nn < 16:
                        ps0 = jnp.where(lane_iota == nn, p, ps0)
                    else:
                        ps1 = jnp.where(lane_iota == (nn - 16), p, ps1)
                fo = kk * PSW
                psb_v[jr, fo // 128, pl.ds(fo % 128, L)] = ps0
                psb_v[jr, fo // 128, pl.ds(fo % 128 + L, L)] = ps1

        def outer(ch, _):
            jd = lax.rem(ch, NBUF)
            for j in range(NBUF):
                @pl.when(jd == j)
                def _():
                    gather_wait(j)

                    @pl.when(ch >= NBUF)
                    def _():
                        ps_wait(j)
            process(jd, jd, ch)
            for j in range(NBUF):
                @pl.when(jd == j)
                def _():
                    ps_start(j, ch)

                    @pl.when(ch + NBUF < NCH)
                    def _():
                        gather_start(j, ch + NBUF)
            return 0

        lax.fori_loop(0, NCH, outer, 0)
        for j in range(NBUF):
            ps_wait(j)

    return sc_scores


def _loss_body(ps_ref, out_ref):
    x = ps_ref[...]
    col = lax.broadcasted_iota(jnp.int32, x.shape, 1)
    xm = jnp.where(col < NIN, x, -1e30)
    m = jnp.max(xm, axis=1, keepdims=True)
    se = jnp.sum(jnp.exp(xm - m), axis=1, keepdims=True)
    lse = m + jnp.log(se)
    out_ref[...] = jnp.mean(lse - x[:, 0:1]).reshape(1, 1)


def kernel(targets, contexts, negtives, wordemb):
    B = targets.shape[0]
    V = wordemb.shape[0]
    idx_all = jnp.concatenate(
        [
            targets.astype(jnp.int32).reshape(B, 1),
            negtives.astype(jnp.int32).reshape(B, -1),
            contexts.astype(jnp.int32).reshape(B, -1),
            jnp.zeros((B, RPI - NIN - NCTX), jnp.int32),
        ],
        axis=1,
    ).reshape(B * RPI // 128, 128)
    table2 = wordemb.reshape(V // 2, 2 * H)
    ps = _make_sc_scores(B, V)(idx_all, table2).reshape(B, PSW)
    loss = pl.pallas_call(
        _loss_body,
        out_shape=jax.ShapeDtypeStruct((1, 1), jnp.float32),
    )(ps)
    return loss[0, 0]


# speedup vs baseline: 4.4422x; 1.5244x over previous
import functools

import jax
import jax.numpy as jnp
from jax import lax
from jax.experimental import pallas as pl
from jax.experimental.pallas import tpu as pltpu
from jax.experimental.pallas import tpu_sc as plsc

H = 64
NIN = 26
NCTX = 20
RPI = 48
C = 4
ROWS = RPI * C
NBUF = 2
PSW = 32
L = 16
NQ = H // L


@functools.lru_cache(maxsize=None)
def _make_sc_scores(B: int, V: int):
    info = plsc.get_sparse_core_info()
    NC, NS = info.num_cores, info.num_subcores
    NW = NC * NS
    assert B % (NW * C) == 0
    BPW = B // NW
    NCH = BPW // C

    mesh = plsc.VectorSubcoreMesh(core_axis_name="c", subcore_axis_name="s")

    @functools.partial(
        pl.kernel,
        mesh=mesh,
        compiler_params=pltpu.CompilerParams(
            needs_layout_passes=False, use_tc_tiling_on_sc=False),
        out_type=jax.ShapeDtypeStruct((B, PSW), jnp.float32),
        scratch_types=[
            pltpu.VMEM((NCH, ROWS), jnp.int32),
            pltpu.VMEM((NBUF, ROWS, H), jnp.float32),
            pltpu.VMEM((BPW, PSW), jnp.float32),
            pltpu.SemaphoreType.DMA,
            pltpu.SemaphoreType.DMA,
        ],
    )
    def sc_scores(idx_hbm, table_hbm, ps_hbm, idx_v, rows_v, ps_v, s0, s1):
        sems = [s0, s1]
        wid = lax.axis_index("s") * NC + lax.axis_index("c")
        pltpu.sync_copy(idx_hbm.at[pl.ds(wid * NCH, NCH)], idx_v)

        def gather_start(j, ch):
            def issue(t, _):
                iv = idx_v[ch, pl.ds(t * L, L)]
                for e in range(L):
                    pltpu.make_async_copy(
                        table_hbm.at[iv[e]],
                        rows_v.at[j, t * L + e], sems[j]).start()
                return 0

            lax.fori_loop(0, ROWS // L, issue, 0)

        def gather_wait(j):
            pltpu.make_async_copy(
                table_hbm.at[pl.ds(0, ROWS)], rows_v.at[j], sems[j]).wait()

        for j in range(NBUF):
            gather_start(j, j)

        lane_iota = lax.iota(jnp.int32, 16)

        def process(jd, ch):
            for k in range(C):
                base = k * RPI
                iv = [idx_v[ch, pl.ds(base + 16 * t, 16)]
                      for t in range(RPI // L)]
                mv = [jnp.where(v > 0, 1.0, 0.0) for v in iv]

                ctx = [jnp.zeros((L,), jnp.float32)] * NQ
                for w in range(NCTX):
                    rr = NIN + w
                    m = mv[rr // 16][rr % 16]
                    for q in range(NQ):
                        ctx[q] = ctx[q] + rows_v[jd, base + rr, pl.ds(q * L, L)] * m
                ctx = [cq * (1.0 / NCTX) for cq in ctx]

                ps0 = jnp.zeros((L,), jnp.float32)
                ps1 = jnp.zeros((L,), jnp.float32)
                for nn in range(NIN):
                    r = base + nn
                    t = rows_v[jd, r, pl.ds(0, L)] * ctx[0]
                    for q in range(1, NQ):
                        t = t + rows_v[jd, r, pl.ds(q * L, L)] * ctx[q]
                    p = jnp.sum(t) * mv[nn // 16][nn % 16]
                    if nn < 16:
                        ps0 = jnp.where(lane_iota == nn, p, ps0)
                    else:
                        ps1 = jnp.where(lane_iota == (nn - 16), p, ps1)
                il = ch * C + k
                ps_v[il, pl.ds(0, L)] = ps0
                ps_v[il, pl.ds(L, L)] = ps1

        def outer(ch, _):
            jd = lax.rem(ch, NBUF)
            for j in range(NBUF):
                @pl.when(jd == j)
                def _():
                    gather_wait(j)
            process(jd, ch)

            @pl.when(ch + NBUF < NCH)
            def _():
                for j in range(NBUF):
                    @pl.when(jd == j)
                    def _():
                        gather_start(j, ch + NBUF)
            return 0

        lax.fori_loop(0, NCH, outer, 0)
        pltpu.sync_copy(ps_v, ps_hbm.at[pl.ds(wid * BPW, BPW)])

    return sc_scores


def _loss_body(ps_ref, out_ref):
    x = ps_ref[...]
    col = lax.broadcasted_iota(jnp.int32, x.shape, 1)
    xm = jnp.where(col < NIN, x, -1e30)
    m = jnp.max(xm, axis=1, keepdims=True)
    se = jnp.sum(jnp.exp(xm - m), axis=1, keepdims=True)
    lse = m + jnp.log(se)
    out_ref[...] = jnp.mean(lse - x[:, 0:1]).reshape(1, 1)


def kernel(targets, contexts, negtives, wordemb):
    B = targets.shape[0]
    V = wordemb.shape[0]
    idx_all = jnp.concatenate(
        [
            targets.astype(jnp.int32).reshape(B, 1),
            negtives.astype(jnp.int32).reshape(B, -1),
            contexts.astype(jnp.int32).reshape(B, -1),
            jnp.zeros((B, RPI - NIN - NCTX), jnp.int32),
        ],
        axis=1,
    ).reshape(B // C, ROWS)
    ps = _make_sc_scores(B, V)(idx_all, wordemb)
    loss = pl.pallas_call(
        _loss_body,
        out_shape=jax.ShapeDtypeStruct((1, 1), jnp.float32),
    )(ps)
    return loss[0, 0]
